# Initial kernel scaffold; baseline (speedup 1.0000x reference)
import jax
import jax.numpy as jnp
from jax.experimental import pallas as pl


def kernel(x, h, c, i_gate, f_gate, W_l0, b_l0, W_r0, W_l1, b_l1, W_r1, W_lin, b_lin, edge_index):
    raise NotImplementedError("write your pallas kernel here")


# speedup vs baseline: 4.4423x; 4.4423x over previous
import functools

import jax
import jax.numpy as jnp
from jax import lax
from jax.experimental import pallas as pl
from jax.experimental.pallas import tpu as pltpu
from jax.experimental.pallas import tpu_sc as plsc

N = 10000
E = 320000
DIN = 128
DOUT = 256

NC = 2
NS = 16
NW = NC * NS

K = 128
NCHUNK = 79
EPT = K * NCHUNK
E_PAD = EPT * NW
NPAD = 10240
RPT = NPAD // NS


def _sc_segment_sum(src_p, dst_p, x):
    mesh = plsc.VectorSubcoreMesh(core_axis_name="c", subcore_axis_name="s",
                                  num_cores=NC, num_subcores=NS)

    @functools.partial(
        pl.kernel,
        out_type=(
            jax.ShapeDtypeStruct((NC * NPAD, DIN), jnp.float32),
            jax.ShapeDtypeStruct((NW, NPAD), jnp.float32),
        ),
        mesh=mesh,
        scratch_types=[
            pltpu.VMEM((K,), jnp.int32),
            pltpu.VMEM((K,), jnp.int32),
            pltpu.VMEM((K, DIN), jnp.float32),
            pltpu.VMEM((NPAD,), jnp.float32),
            pltpu.VMEM_SHARED((NPAD, DIN), jnp.float32),
            pltpu.SemaphoreType.DMA,
        ],
        compiler_params=pltpu.CompilerParams(needs_layout_passes=False),
    )
    def seg(src_hbm, dst_hbm, x_hbm, iota_hbm, zrow_hbm, zdeg_hbm,
            agg_out, deg_out,
            src_v, dst_v, rows_v, deg_v, agg_s, sem):
        cid = lax.axis_index("c")
        sid = lax.axis_index("s")
        wid = sid * NC + cid

        pltpu.sync_copy(zrow_hbm, rows_v)
        pltpu.sync_copy(zdeg_hbm, deg_v)

        @pl.loop(0, RPT // K)
        def _zero(m):
            pltpu.sync_copy(iota_hbm.at[pl.ds(sid * RPT + m * K, K)], src_v)
            pltpu.sync_copy(rows_v, agg_s.at[src_v])

        plsc.subcore_barrier()

        base = wid * EPT
        one16 = jnp.full((16,), 1.0, jnp.float32)

        @pl.loop(0, NCHUNK)
        def _chunk(j):
            off = base + j * K
            pltpu.sync_copy(src_hbm.at[pl.ds(off, K)], src_v)
            pltpu.sync_copy(dst_hbm.at[pl.ds(off, K)], dst_v)
            pltpu.async_copy(x_hbm.at[src_v], rows_v, sem).wait()
            pltpu.sync_copy(rows_v, agg_s.at[dst_v], add=True)
            for g in range(K // 16):
                idx16 = dst_v[pl.ds(g * 16, 16)]
                plsc.addupdate_scatter(deg_v, [idx16], one16)

        plsc.subcore_barrier()

        obase = cid * NPAD + sid * RPT

        @pl.loop(0, RPT // K)
        def _pub(m):
            pltpu.sync_copy(iota_hbm.at[pl.ds(sid * RPT + m * K, K)], src_v)
            pltpu.async_copy(agg_s.at[src_v], rows_v, sem).wait()
            pltpu.sync_copy(rows_v, agg_out.at[pl.ds(obase + m * K, K)])

        pltpu.sync_copy(deg_v, deg_out.at[wid])

    return seg(src_p, dst_p, x, jnp.arange(NPAD, dtype=jnp.int32),
               jnp.zeros((K, DIN), jnp.float32),
               jnp.zeros((NPAD,), jnp.float32))


RB = 1000


def _tc_body(agg0, agg1, degt, x, c, i_g, f_g, wl, wr, wlin, bl, blin, out):
    deg = jnp.sum(degt[...], axis=1, keepdims=True)
    denom = jnp.maximum(deg, 1.0)
    mean = (agg0[...] + agg1[...]) / denom
    t = (jnp.dot(mean, wl[...], preferred_element_type=jnp.float32)
         + jnp.dot(x[...], wr[...], preferred_element_type=jnp.float32)
         + bl[...])
    t = jnp.dot(t, wlin[...], preferred_element_type=jnp.float32) + blin[...]
    t = jnp.tanh(t)
    out[...] = f_g[...] * c[...] + i_g[...] * t


def _tc_tail(agg0, agg1, degt, x, c, i_gate, f_gate, W_l1, W_r1, W_lin, b_l1, b_lin):
    row = lambda i: (i, 0)
    full = lambda i: (0, 0)
    return pl.pallas_call(
        _tc_body,
        grid=(N // RB,),
        in_specs=[
            pl.BlockSpec((RB, DIN), row),
            pl.BlockSpec((RB, DIN), row),
            pl.BlockSpec((RB, NW), row),
            pl.BlockSpec((RB, DIN), row),
            pl.BlockSpec((RB, DOUT), row),
            pl.BlockSpec((RB, DOUT), row),
            pl.BlockSpec((RB, DOUT), row),
            pl.BlockSpec((DIN, DOUT), full),
            pl.BlockSpec((DIN, DOUT), full),
            pl.BlockSpec((DOUT, DOUT), full),
            pl.BlockSpec((1, DOUT), full),
            pl.BlockSpec((1, DOUT), full),
        ],
        out_specs=pl.BlockSpec((RB, DOUT), row),
        out_shape=jax.ShapeDtypeStruct((N, DOUT), jnp.float32),
    )(agg0, agg1, degt, x, c, i_gate, f_gate, W_l1, W_r1, W_lin, b_l1, b_lin)


def kernel(x, h, c, i_gate, f_gate, W_l0, b_l0, W_r0, W_l1, b_l1, W_r1, W_lin, b_lin, edge_index):
    src = edge_index[0]
    dst = edge_index[1]
    pad = E_PAD - E
    src_p = jnp.concatenate([src, jnp.zeros((pad,), jnp.int32)])
    dst_p = jnp.concatenate([dst, jnp.full((pad,), N, jnp.int32)])

    agg_part, deg_part = _sc_segment_sum(src_p, dst_p, x)
    agg0 = agg_part[:N]
    agg1 = agg_part[NPAD:NPAD + N]
    degt = deg_part[:, :N].T

    return _tc_tail(agg0, agg1, degt, x, c, i_gate, f_gate,
                    W_l1, W_r1, W_lin,
                    b_l1.reshape(1, DOUT), b_lin.reshape(1, DOUT))
